# baseline (speedup 1.0000x reference)
import jax
import jax.numpy as jnp
from jax import lax
from jax.experimental import pallas as pl
from jax.experimental.pallas import tpu as pltpu
from jax.experimental.pallas import tpu_sc as plsc

_B = 4096
_L = 50
_E = 32
_N = _B * _L
_V = 1000000
_LBL = 1000
_LBL_PAD = 1024

_NC = 2
_NS = 16
_NW = _NC * _NS
_PER_W = _N // _NW
_CH = 128
_NCH = _PER_W // _CH
_PAIRS = _NCH // 2

_NTC = _V // 128
_KFULL = 244


def _sc_transpose_body(tt_hbm, tail_hbm, out_hbm, abuf0, abuf1, obuf0, obuf1, st0, st1, os0, os1):
    abufs = (abuf0, abuf1)
    obufs = (obuf0, obuf1)
    wid = lax.axis_index("s") * _NC + lax.axis_index("c")
    v16 = lax.iota(jnp.int32, 16)
    pat32 = v16 * 32

    def stage(c, slot, sem):
        off = pl.multiple_of(c * 128, 128)
        for eb in range(4):
            pltpu.async_copy(
                tt_hbm.at[pl.ds(8 * eb, 8), pl.ds(off, 128)],
                abufs[slot].at[pl.ds(8 * eb, 8)], sem)

    def stage_wait(c, slot, sem):
        off = pl.multiple_of(c * 128, 128)
        for eb in range(4):
            pltpu.make_async_copy(
                tt_hbm.at[pl.ds(8 * eb, 8), pl.ds(off, 128)],
                abufs[slot].at[pl.ds(8 * eb, 8)], sem).wait()

    def shuffle(slot):
        @plsc.parallel_loop(0, 8, step=1, unroll=8)
        def _(g):
            goff = pl.multiple_of(g * 16, 16)
            base = pat32 + g * 512
            for e in range(_E):
                val = abufs[slot][e, pl.ds(goff, 16)]
                plsc.store_scatter(obufs[slot], [base + e], val)

    def out_descr(c, slot, sem):
        return pltpu.make_async_copy(
            obufs[slot], out_hbm.at[pl.ds(pl.multiple_of(c * 4096, 8), 4096)],
            sem)

    def col(k):
        return wid + 32 * k

    stage(col(0), 0, st0)
    stage(col(1), 1, st1)

    def half(kk, k, slot, stx, osx):
        stage_wait(col(k), slot, stx)

        @pl.when(kk >= 1)
        def _():
            out_descr(col(k - 2), slot, osx).wait()

        shuffle(slot)
        out_descr(col(k), slot, osx).start()

        @pl.when(k + 2 < _KFULL)
        def _():
            stage(col(k + 2), slot, stx)

    def step(kk, carry):
        half(kk, kk * 2, 0, st0, os0)
        half(kk, kk * 2 + 1, 1, st1, os1)
        return carry

    lax.fori_loop(0, _KFULL // 2, step, 0, unroll=False)
    out_descr(col(_KFULL - 2), 0, os0).wait()
    out_descr(col(_KFULL - 1), 1, os1).wait()

    @pl.when(wid < 4)
    def _():
        c = 7808 + wid
        stage(c, 0, st0)
        stage_wait(c, 0, st0)
        shuffle(0)
        out_descr(c, 0, os0).start()
        out_descr(c, 0, os0).wait()

    @pl.when(wid == 4)
    def _():
        pltpu.sync_copy(tail_hbm, obuf0.at[pl.ds(0, 2048)])
        pltpu.sync_copy(obuf0.at[pl.ds(0, 2048)],
                        out_hbm.at[pl.ds(7812 * 4096, 2048)])


def _sc_table_rowmajor(tt, tail):
    mesh = plsc.VectorSubcoreMesh(core_axis_name="c", subcore_axis_name="s")
    fn = pl.kernel(
        _sc_transpose_body,
        out_type=jax.ShapeDtypeStruct((_V * _E,), jnp.float32),
        mesh=mesh,
        scratch_types=[
            pltpu.VMEM((32, 128), jnp.float32),
            pltpu.VMEM((32, 128), jnp.float32),
            pltpu.VMEM((4096,), jnp.float32),
            pltpu.VMEM((4096,), jnp.float32),
            pltpu.SemaphoreType.DMA,
            pltpu.SemaphoreType.DMA,
            pltpu.SemaphoreType.DMA,
            pltpu.SemaphoreType.DMA,
        ],
        compiler_params=pltpu.CompilerParams(
            use_tc_tiling_on_sc=True, needs_layout_passes=False,
            disable_bounds_checks=True),
    )
    return fn(tt, tail).reshape(_V, _E)


def _sc_gather_body(idx_hbm, table_hbm, out_hbm, idx_v, rows_v, gs0, gs1, ss0, ss1):
    wid = lax.axis_index("s") * _NC + lax.axis_index("c")
    base = wid * _PER_W

    pltpu.sync_copy(idx_hbm.at[pl.ds(base, _PER_W)], idx_v)

    def idx_slice(j):
        return idx_v.at[pl.ds(pl.multiple_of(j * _CH, _CH), _CH)]

    def fire_gather(j, buf, sem):
        pltpu.async_copy(table_hbm.at[idx_slice(j)], rows_v.at[buf], sem)

    def wait_gather(j, buf, sem):
        pltpu.make_async_copy(table_hbm.at[idx_slice(j)], rows_v.at[buf], sem).wait()

    def store_descr(j, buf, sem):
        row0 = pl.multiple_of(base + j * _CH, _CH)
        return pltpu.make_async_copy(
            rows_v.at[buf], out_hbm.at[pl.ds(row0, _CH)], sem)

    fire_gather(0, 0, gs0)
    fire_gather(1, 1, gs1)

    def step(jj, carry):
        j0 = jj * 2
        j1 = j0 + 1
        wait_gather(j0, 0, gs0)
        store_descr(j0, 0, ss0).start()
        wait_gather(j1, 1, gs1)
        store_descr(j1, 1, ss1).start()

        @pl.when(jj < _PAIRS - 1)
        def _():
            store_descr(j0, 0, ss0).wait()
            fire_gather(j0 + 2, 0, gs0)
            store_descr(j1, 1, ss1).wait()
            fire_gather(j1 + 2, 1, gs1)

        return carry

    lax.fori_loop(0, _PAIRS, step, 0)
    store_descr(_NCH - 2, 0, ss0).wait()
    store_descr(_NCH - 1, 1, ss1).wait()


def _sc_gather(idx1d, table):
    mesh = plsc.VectorSubcoreMesh(core_axis_name="c", subcore_axis_name="s")
    fn = pl.kernel(
        _sc_gather_body,
        out_type=jax.ShapeDtypeStruct((_N, _E), jnp.float32),
        mesh=mesh,
        scratch_types=[
            pltpu.VMEM((_PER_W,), jnp.int32),
            pltpu.VMEM((2, _CH, _E), jnp.float32),
            pltpu.SemaphoreType.DMA,
            pltpu.SemaphoreType.DMA,
            pltpu.SemaphoreType.DMA,
            pltpu.SemaphoreType.DMA,
        ],
        compiler_params=pltpu.CompilerParams(use_tc_tiling_on_sc=False),
    )
    return fn(idx1d, table)


def _head_body(flat_ref, w_ref, bias_ref, out_ref):
    a = flat_ref[...].astype(jnp.bfloat16)
    logits = lax.dot_general(
        w_ref[...], a, (((1,), (1,)), ((), ())),
        preferred_element_type=jnp.float32,
    )
    logits = logits + bias_ref[...]
    m = jnp.max(logits, axis=0, keepdims=True)
    e = jnp.exp(logits - m)
    s = jnp.sum(e, axis=0, keepdims=True)
    res = logits - m - jnp.log(s)
    out_ref[...] = res[: _LBL, :]


def _tc_head(flat, wpad, bias):
    bs = 512
    grid = (_B // bs,)
    return pl.pallas_call(
        _head_body,
        grid=grid,
        in_specs=[
            pl.BlockSpec((bs, _L * _E), lambda i: (i, 0)),
            pl.BlockSpec((_LBL_PAD, _L * _E), lambda i: (0, 0)),
            pl.BlockSpec((_LBL_PAD, 1), lambda i: (0, 0)),
        ],
        out_specs=pl.BlockSpec((_LBL, bs), lambda i: (0, i)),
        out_shape=jax.ShapeDtypeStruct((_LBL, _B), jnp.float32),
    )(flat, wpad, bias)


def kernel(x, table, W, b):
    idx1d = x.reshape(_N).astype(jnp.int32)
    tail = table[_V - 64:, :].reshape(64 * _E)
    tbl_rm = _sc_table_rowmajor(table.T, tail)
    emb = _sc_gather(idx1d, tbl_rm)
    flat = emb.reshape(_B, _L * _E)
    wpad = jnp.pad(W, ((0, _LBL_PAD - _LBL), (0, 0))).astype(jnp.bfloat16)
    bias = jnp.concatenate([b, jnp.full((_LBL_PAD - _LBL,), -1e30, b.dtype)])
    out_t = _tc_head(flat, wpad, bias.reshape(_LBL_PAD, 1))
    return out_t.T


# speedup vs baseline: 2.5673x; 2.5673x over previous
import jax
import jax.numpy as jnp
from jax import lax
from jax.experimental import pallas as pl
from jax.experimental.pallas import tpu as pltpu
from jax.experimental.pallas import tpu_sc as plsc

_B = 4096
_L = 50
_E = 32
_N = _B * _L
_V = 1000000
_LBL = 1000
_LBL_PAD = 1024

_NC = 2
_NS = 16
_NW = _NC * _NS
_PER_W = _N // _NW
_CH = 128
_NCH = _PER_W // _CH
_PAIRS = _NCH // 2

_NTC = _V // 128
_KFULL = 244


def _sc_transpose_body(tt_hbm, tail_hbm, out_hbm, abuf0, abuf1, obuf0, obuf1, st0, st1, os0, os1):
    abufs = (abuf0, abuf1)
    obufs = (obuf0, obuf1)
    wid = lax.axis_index("s") * _NC + lax.axis_index("c")
    v16 = lax.iota(jnp.int32, 16)
    pat32 = v16 * 32

    def stage(c, slot, sem):
        off = pl.multiple_of(c * 128, 128)
        for eb in range(4):
            pltpu.async_copy(
                tt_hbm.at[pl.ds(8 * eb, 8), pl.ds(off, 128)],
                abufs[slot].at[pl.ds(8 * eb, 8)], sem)

    def stage_wait(c, slot, sem):
        off = pl.multiple_of(c * 128, 128)
        for eb in range(4):
            pltpu.make_async_copy(
                tt_hbm.at[pl.ds(8 * eb, 8), pl.ds(off, 128)],
                abufs[slot].at[pl.ds(8 * eb, 8)], sem).wait()

    def shuffle(slot):
        @plsc.parallel_loop(0, 256, step=1, unroll=8)
        def _(m):
            e0 = m >> 3
            g = m & 7
            e_vec = (e0 + v16) & 31
            val = plsc.load_gather(abufs[slot], [e_vec, g * 16 + v16])
            plsc.store_scatter(obufs[slot], [g * 512 + pat32 + e_vec], val)

    def out_descr(c, slot, sem):
        return pltpu.make_async_copy(
            obufs[slot], out_hbm.at[pl.ds(pl.multiple_of(c * 4096, 8), 4096)],
            sem)

    def col(k):
        return wid + 32 * k

    stage(col(0), 0, st0)
    stage(col(1), 1, st1)

    def half(kk, k, slot, stx, osx):
        stage_wait(col(k), slot, stx)

        @pl.when(kk >= 1)
        def _():
            out_descr(col(k - 2), slot, osx).wait()

        shuffle(slot)
        out_descr(col(k), slot, osx).start()

        @pl.when(k + 2 < _KFULL)
        def _():
            stage(col(k + 2), slot, stx)

    def step(kk, carry):
        half(kk, kk * 2, 0, st0, os0)
        half(kk, kk * 2 + 1, 1, st1, os1)
        return carry

    lax.fori_loop(0, _KFULL // 2, step, 0, unroll=False)
    out_descr(col(_KFULL - 2), 0, os0).wait()
    out_descr(col(_KFULL - 1), 1, os1).wait()

    @pl.when(wid < 4)
    def _():
        c = 7808 + wid
        stage(c, 0, st0)
        stage_wait(c, 0, st0)
        shuffle(0)
        out_descr(c, 0, os0).start()
        out_descr(c, 0, os0).wait()

    @pl.when(wid == 4)
    def _():
        pltpu.sync_copy(tail_hbm, obuf0.at[pl.ds(0, 2048)])
        pltpu.sync_copy(obuf0.at[pl.ds(0, 2048)],
                        out_hbm.at[pl.ds(7812 * 4096, 2048)])


def _sc_table_rowmajor(tt, tail):
    mesh = plsc.VectorSubcoreMesh(core_axis_name="c", subcore_axis_name="s")
    fn = pl.kernel(
        _sc_transpose_body,
        out_type=jax.ShapeDtypeStruct((_V * _E,), jnp.float32),
        mesh=mesh,
        scratch_types=[
            pltpu.VMEM((32, 128), jnp.float32),
            pltpu.VMEM((32, 128), jnp.float32),
            pltpu.VMEM((4096,), jnp.float32),
            pltpu.VMEM((4096,), jnp.float32),
            pltpu.SemaphoreType.DMA,
            pltpu.SemaphoreType.DMA,
            pltpu.SemaphoreType.DMA,
            pltpu.SemaphoreType.DMA,
        ],
        compiler_params=pltpu.CompilerParams(
            use_tc_tiling_on_sc=True, needs_layout_passes=False,
            disable_bounds_checks=True),
    )
    return fn(tt, tail).reshape(_V, _E)


def _sc_gather_body(idx_hbm, table_hbm, out_hbm, idx_v, rows_v, gs0, gs1, ss0, ss1):
    wid = lax.axis_index("s") * _NC + lax.axis_index("c")
    base = wid * _PER_W

    pltpu.sync_copy(idx_hbm.at[pl.ds(base, _PER_W)], idx_v)

    def idx_slice(j):
        return idx_v.at[pl.ds(pl.multiple_of(j * _CH, _CH), _CH)]

    def fire_gather(j, buf, sem):
        pltpu.async_copy(table_hbm.at[idx_slice(j)], rows_v.at[buf], sem)

    def wait_gather(j, buf, sem):
        pltpu.make_async_copy(table_hbm.at[idx_slice(j)], rows_v.at[buf], sem).wait()

    def store_descr(j, buf, sem):
        row0 = pl.multiple_of(base + j * _CH, _CH)
        return pltpu.make_async_copy(
            rows_v.at[buf], out_hbm.at[pl.ds(row0, _CH)], sem)

    fire_gather(0, 0, gs0)
    fire_gather(1, 1, gs1)

    def step(jj, carry):
        j0 = jj * 2
        j1 = j0 + 1
        wait_gather(j0, 0, gs0)
        store_descr(j0, 0, ss0).start()
        wait_gather(j1, 1, gs1)
        store_descr(j1, 1, ss1).start()

        @pl.when(jj < _PAIRS - 1)
        def _():
            store_descr(j0, 0, ss0).wait()
            fire_gather(j0 + 2, 0, gs0)
            store_descr(j1, 1, ss1).wait()
            fire_gather(j1 + 2, 1, gs1)

        return carry

    lax.fori_loop(0, _PAIRS, step, 0)
    store_descr(_NCH - 2, 0, ss0).wait()
    store_descr(_NCH - 1, 1, ss1).wait()


def _sc_gather(idx1d, table):
    mesh = plsc.VectorSubcoreMesh(core_axis_name="c", subcore_axis_name="s")
    fn = pl.kernel(
        _sc_gather_body,
        out_type=jax.ShapeDtypeStruct((_N, _E), jnp.float32),
        mesh=mesh,
        scratch_types=[
            pltpu.VMEM((_PER_W,), jnp.int32),
            pltpu.VMEM((2, _CH, _E), jnp.float32),
            pltpu.SemaphoreType.DMA,
            pltpu.SemaphoreType.DMA,
            pltpu.SemaphoreType.DMA,
            pltpu.SemaphoreType.DMA,
        ],
        compiler_params=pltpu.CompilerParams(use_tc_tiling_on_sc=False),
    )
    return fn(idx1d, table)


def _head_body(flat_ref, w_ref, bias_ref, out_ref):
    a = flat_ref[...].astype(jnp.bfloat16)
    logits = lax.dot_general(
        w_ref[...], a, (((1,), (1,)), ((), ())),
        preferred_element_type=jnp.float32,
    )
    logits = logits + bias_ref[...]
    m = jnp.max(logits, axis=0, keepdims=True)
    e = jnp.exp(logits - m)
    s = jnp.sum(e, axis=0, keepdims=True)
    res = logits - m - jnp.log(s)
    out_ref[...] = res[: _LBL, :]


def _tc_head(flat, wpad, bias):
    bs = 512
    grid = (_B // bs,)
    return pl.pallas_call(
        _head_body,
        grid=grid,
        in_specs=[
            pl.BlockSpec((bs, _L * _E), lambda i: (i, 0)),
            pl.BlockSpec((_LBL_PAD, _L * _E), lambda i: (0, 0)),
            pl.BlockSpec((_LBL_PAD, 1), lambda i: (0, 0)),
        ],
        out_specs=pl.BlockSpec((_LBL, bs), lambda i: (0, i)),
        out_shape=jax.ShapeDtypeStruct((_LBL, _B), jnp.float32),
    )(flat, wpad, bias)


def kernel(x, table, W, b):
    idx1d = x.reshape(_N).astype(jnp.int32)
    tail = table[_V - 64:, :].reshape(64 * _E)
    tbl_rm = _sc_table_rowmajor(table.T, tail)
    emb = _sc_gather(idx1d, tbl_rm)
    flat = emb.reshape(_B, _L * _E)
    wpad = jnp.pad(W, ((0, _LBL_PAD - _LBL), (0, 0))).astype(jnp.bfloat16)
    bias = jnp.concatenate([b, jnp.full((_LBL_PAD - _LBL,), -1e30, b.dtype)])
    out_t = _tc_head(flat, wpad, bias.reshape(_LBL_PAD, 1))
    return out_t.T
